# Initial kernel scaffold; baseline (speedup 1.0000x reference)
import jax
import jax.numpy as jnp
from jax.experimental import pallas as pl


def kernel(x):
    raise NotImplementedError("write your pallas kernel here")


# speedup vs baseline: 11.1061x; 11.1061x over previous
import functools

import jax
import jax.numpy as jnp
from jax.experimental import pallas as pl

_K = 512
_ROWS_PER_BLOCK = 8


def _topk_mask_kernel(x_ref, o_ref, *, k):
    xb = x_ref[...]
    i = jax.lax.bitcast_convert_type(xb, jnp.int32)
    s = i ^ ((i >> 31) & jnp.int32(0x7FFFFFFF))
    c = jnp.sum((s >= 0).astype(jnp.int32), axis=1, keepdims=True)
    t = jnp.where(c >= k, jnp.int32(0), jnp.int32(-2147483648))
    for b in range(30, -1, -1):
        cand = t | jnp.int32(1 << b)
        c = jnp.sum((s >= cand).astype(jnp.int32), axis=1, keepdims=True)
        t = jnp.where(c >= k, cand, t)
    o_ref[...] = jnp.where(s >= t, xb, jnp.float32(0.0))


@jax.jit
def kernel(x):
    m, n = x.shape
    r = _ROWS_PER_BLOCK
    return pl.pallas_call(
        functools.partial(_topk_mask_kernel, k=_K),
        out_shape=jax.ShapeDtypeStruct(x.shape, x.dtype),
        grid=(m // r,),
        in_specs=[pl.BlockSpec((r, n), lambda i: (i, 0))],
        out_specs=pl.BlockSpec((r, n), lambda i: (i, 0)),
    )(x)


# speedup vs baseline: 40.3586x; 3.6339x over previous
import functools

import jax
import jax.numpy as jnp
from jax.experimental import pallas as pl

_K = 512
_ROWS_PER_BLOCK = 32


def _row_count(mask):
    v = mask.astype(jnp.int32)
    n = v.shape[1]
    while n > 128:
        n //= 2
        v = v[:, :n] + v[:, n:]
    return jnp.sum(v, axis=1, keepdims=True)


def _topk_mask_kernel(x_ref, o_ref, *, k):
    xb = x_ref[...]
    i = jax.lax.bitcast_convert_type(xb, jnp.int32)
    s = i ^ ((i >> 31) & jnp.int32(0x7FFFFFFF))
    c = _row_count(s >= 0)
    t = jnp.where(c >= k, jnp.int32(0), jnp.int32(-2147483648))
    for b in range(30, -1, -1):
        cand = t | jnp.int32(1 << b)
        c = _row_count(s >= cand)
        t = jnp.where(c >= k, cand, t)
    o_ref[...] = jnp.where(s >= t, xb, jnp.float32(0.0))


@jax.jit
def kernel(x):
    m, n = x.shape
    r = _ROWS_PER_BLOCK
    return pl.pallas_call(
        functools.partial(_topk_mask_kernel, k=_K),
        out_shape=jax.ShapeDtypeStruct(x.shape, x.dtype),
        grid=(m // r,),
        in_specs=[pl.BlockSpec((r, n), lambda i: (i, 0))],
        out_specs=pl.BlockSpec((r, n), lambda i: (i, 0)),
    )(x)
